# baseline (speedup 1.0000x reference)
import functools

import jax
import jax.numpy as jnp
from jax import lax
from jax.experimental import pallas as pl
from jax.experimental.pallas import tpu as pltpu
from jax.experimental.pallas import tpu_sc as plsc

T_OUT = 2048
_C = 16
_CZ = 16
_NB = 6
_L = 16


@functools.lru_cache(maxsize=None)
def _build_sc_kernel(N, P, D, T, NW, NC):
    rows_w = N // NW
    nchunks = rows_w // _C
    pad_w = P // NW
    pchunks = pad_w // _CZ

    mesh = plsc.VectorSubcoreMesh(core_axis_name="c", subcore_axis_name="s")

    @functools.partial(
        pl.kernel,
        mesh=mesh,
        compiler_params=pltpu.CompilerParams(needs_layout_passes=False),
        out_type=jax.ShapeDtypeStruct((N + P, D), jnp.float32),
        scratch_types=[
            pltpu.VMEM((T,), jnp.int32),
            pltpu.VMEM((32,), jnp.int32),
            pltpu.VMEM((_L,), jnp.int32),
            pltpu.VMEM((nchunks, _C), jnp.int32),
            pltpu.VMEM((pchunks, _CZ), jnp.int32),
            pltpu.VMEM((_C, D), jnp.float32),
            pltpu.VMEM((_C, D), jnp.float32),
            pltpu.VMEM((_C, D), jnp.float32),
            pltpu.VMEM((_C, D), jnp.float32),
            pltpu.VMEM((_C, D), jnp.float32),
            pltpu.VMEM((_C, D), jnp.float32),
            pltpu.VMEM((_CZ, D), jnp.float32),
            pltpu.SemaphoreType.DMA,
            pltpu.SemaphoreType.DMA,
            pltpu.SemaphoreType.DMA,
            pltpu.SemaphoreType.DMA,
            pltpu.SemaphoreType.DMA,
            pltpu.SemaphoreType.DMA,
            pltpu.SemaphoreType.DMA,
            pltpu.SemaphoreType.DMA,
            pltpu.SemaphoreType.DMA,
            pltpu.SemaphoreType.DMA,
            pltpu.SemaphoreType.DMA,
            pltpu.SemaphoreType.DMA,
            pltpu.SemaphoreType.DMA,
            pltpu.SemaphoreType.DMA,
        ],
    )
    def k(x_hbm, prefix_hbm, cumpad_hbm, len_hbm, zsrc_hbm, out_hbm,
          prefix_v, cumpad_v, len_v, sidx_v, zidx_v,
          buf0, buf1, buf2, buf3, buf4, buf5, zero_v,
          l0, l1, l2, l3, l4, l5, s0, s1, s2, s3, s4, s5, zsem, tsem):
        bufs = (buf0, buf1, buf2, buf3, buf4, buf5)
        lsem = (l0, l1, l2, l3, l4, l5)
        ssem = (s0, s1, s2, s3, s4, s5)
        wid = lax.axis_index("s") * NC + lax.axis_index("c")
        base = wid * rows_w
        zbase = wid * pad_w
        lane = jnp.arange(_L, dtype=jnp.int32)

        def load(i, b, sem_i):
            return pltpu.make_async_copy(
                x_hbm.at[pl.ds(base + i * _C, _C), :], bufs[b], lsem[sem_i])

        def scat(i, b, sem_i):
            return pltpu.make_async_copy(
                bufs[b], out_hbm.at[sidx_v.at[i]], ssem[sem_i])

        tabs = (
            pltpu.make_async_copy(prefix_hbm, prefix_v, tsem),
            pltpu.make_async_copy(cumpad_hbm, cumpad_v, tsem),
            pltpu.make_async_copy(len_hbm, len_v, tsem),
            pltpu.make_async_copy(zsrc_hbm, zero_v, tsem),
        )

        for b in range(_NB):
            load(b, b, b).start()
        for t_ in tabs:
            t_.start()
        for t_ in tabs:
            t_.wait()

        def gen_z(i, carry):
            for h in range(_CZ // _L):
                j = lane + (zbase + i * _CZ + h * _L)
                lo = jnp.zeros((_L,), jnp.int32)
                for bit in (16, 8, 4, 2, 1):
                    cand = lo | bit
                    cm = plsc.load_gather(cumpad_v, [cand])
                    lo = jnp.where(cm <= j, cand, lo)
                lb = plsc.load_gather(len_v, [lo])
                cp = plsc.load_gather(cumpad_v, [lo])
                zidx_v[i, pl.ds(h * _L, _L)] = lo * T + lb + (j - cp)
            pltpu.async_copy(zero_v, out_hbm.at[zidx_v.at[i]], zsem)
            return carry

        lax.fori_loop(0, pchunks, gen_z, 0)

        def gen_s(i, carry):
            for h in range(_C // _L):
                p = lane + (base + i * _C + h * _L)
                lo = jnp.zeros((_L,), jnp.int32)
                for bit in (1024, 512, 256, 128, 64, 32, 16, 8, 4, 2, 1):
                    cand = lo | bit
                    pm = plsc.load_gather(prefix_v, [cand])
                    lo = jnp.where(pm <= p, cand, lo)
                pt = plsc.load_gather(prefix_v, [lo])
                sidx_v[i, pl.ds(h * _L, _L)] = (p - pt) * T + lo
            return carry

        lax.fori_loop(0, nchunks, gen_s, 0)

        def body(i, carry):
            for b in range(_NB):
                c = (b + 1) % _NB

                @pl.when(i % _NB == b)
                def _(b=b, c=c):
                    load(i, b, b).wait()
                    scat(i, b, b).start()

                    @pl.when(i + 1 < nchunks)
                    def _(b=b, c=c):
                        @pl.when(i >= _NB - 1)
                        def _(c=c):
                            scat(i - (_NB - 1), c, c).wait()

                            load(i + 1, c, c).start()

            return carry

        lax.fori_loop(0, nchunks, body, 0)

        for j in range(nchunks - _NB, nchunks):
            scat(j, j % _NB, j % _NB).wait()

        def zdrain(j, carry):
            pltpu.make_async_copy(zero_v, out_hbm.at[zidx_v.at[j]],
                                  zsem).wait()
            return carry

        lax.fori_loop(0, pchunks, zdrain, 0)

    return k


def kernel(x, lengths):
    N, D = x.shape
    B = lengths.shape[0]
    T = T_OUT
    P = B * T - N

    info = plsc.get_sparse_core_info()
    NC, NS = info.num_cores, info.num_subcores
    NW = NC * NS

    t = jnp.arange(T, dtype=jnp.int32)
    lens32 = lengths.astype(jnp.int32)
    prefix = jnp.sum(jnp.minimum(lens32[None, :], t[:, None]),
                     axis=1).astype(jnp.int32)
    i32 = jnp.arange(32, dtype=jnp.int32)
    cumpad = jnp.where(
        i32 > B,
        jnp.iinfo(jnp.int32).max,
        jnp.sum(jnp.where(jnp.arange(B, dtype=jnp.int32)[None, :]
                          < i32[:, None],
                          T - lens32[None, :], 0), axis=1)).astype(jnp.int32)
    zsrc = jnp.zeros((_CZ, D), x.dtype)

    k = _build_sc_kernel(N, P, D, T, NW, NC)
    out = k(x, prefix, cumpad, lens32, zsrc)
    return out.reshape(B, T, D)


# speedup vs baseline: 1.1113x; 1.1113x over previous
import functools

import jax
import jax.numpy as jnp
from jax import lax
from jax.experimental import pallas as pl
from jax.experimental.pallas import tpu as pltpu
from jax.experimental.pallas import tpu_sc as plsc

T_OUT = 2048
_C = 32
_CZ = 16
_NB = 3
_L = 16


@functools.lru_cache(maxsize=None)
def _build_sc_kernel(N, P, D, T, NW, NC):
    rows_w = N // NW
    nchunks = rows_w // _C
    pad_w = P // NW
    pchunks = pad_w // _CZ

    mesh = plsc.VectorSubcoreMesh(core_axis_name="c", subcore_axis_name="s")

    @functools.partial(
        pl.kernel,
        mesh=mesh,
        compiler_params=pltpu.CompilerParams(needs_layout_passes=False),
        out_type=jax.ShapeDtypeStruct((N + P, D), jnp.float32),
        scratch_types=[
            pltpu.VMEM((T,), jnp.int32),
            pltpu.VMEM((32,), jnp.int32),
            pltpu.VMEM((_L,), jnp.int32),
            pltpu.VMEM((nchunks, _C), jnp.int32),
            pltpu.VMEM((pchunks, _CZ), jnp.int32),
            pltpu.VMEM((_C, D), jnp.float32),
            pltpu.VMEM((_C, D), jnp.float32),
            pltpu.VMEM((_C, D), jnp.float32),
            pltpu.VMEM((_CZ, D), jnp.float32),
            pltpu.SemaphoreType.DMA,
            pltpu.SemaphoreType.DMA,
            pltpu.SemaphoreType.DMA,
            pltpu.SemaphoreType.DMA,
            pltpu.SemaphoreType.DMA,
            pltpu.SemaphoreType.DMA,
            pltpu.SemaphoreType.DMA,
            pltpu.SemaphoreType.DMA,
        ],
    )
    def k(x_hbm, prefix_hbm, cumpad_hbm, len_hbm, zsrc_hbm, out_hbm,
          prefix_v, cumpad_v, len_v, sidx_v, zidx_v, buf0, buf1, buf2,
          zero_v, l0, l1, l2, s0, s1, s2, zsem, tsem):
        bufs = (buf0, buf1, buf2)
        lsem = (l0, l1, l2)
        ssem = (s0, s1, s2)
        wid = lax.axis_index("s") * NC + lax.axis_index("c")
        base = wid * rows_w
        zbase = wid * pad_w
        lane = jnp.arange(_L, dtype=jnp.int32)

        def load(i, b, sem_i):
            return pltpu.make_async_copy(
                x_hbm.at[pl.ds(base + i * _C, _C), :], bufs[b], lsem[sem_i])

        def scat(i, b, sem_i):
            return pltpu.make_async_copy(
                bufs[b], out_hbm.at[sidx_v.at[i]], ssem[sem_i])

        tabs = (
            pltpu.make_async_copy(prefix_hbm, prefix_v, tsem),
            pltpu.make_async_copy(cumpad_hbm, cumpad_v, tsem),
            pltpu.make_async_copy(len_hbm, len_v, tsem),
            pltpu.make_async_copy(zsrc_hbm, zero_v, tsem),
        )

        for b in range(_NB):
            load(b, b, b).start()
        for t_ in tabs:
            t_.start()
        for t_ in tabs:
            t_.wait()

        def gen_z(i, carry):
            for h in range(_CZ // _L):
                j = lane + (zbase + i * _CZ + h * _L)
                lo = jnp.zeros((_L,), jnp.int32)
                for bit in (16, 8, 4, 2, 1):
                    cand = lo | bit
                    cm = plsc.load_gather(cumpad_v, [cand])
                    lo = jnp.where(cm <= j, cand, lo)
                lb = plsc.load_gather(len_v, [lo])
                cp = plsc.load_gather(cumpad_v, [lo])
                zidx_v[i, pl.ds(h * _L, _L)] = lo * T + lb + (j - cp)
            pltpu.async_copy(zero_v, out_hbm.at[zidx_v.at[i]], zsem)
            return carry

        lax.fori_loop(0, pchunks, gen_z, 0)

        def gen_s(i, carry):
            for h in range(_C // _L):
                p = lane + (base + i * _C + h * _L)
                lo = jnp.zeros((_L,), jnp.int32)
                for bit in (1024, 512, 256, 128, 64, 32, 16, 8, 4, 2, 1):
                    cand = lo | bit
                    pm = plsc.load_gather(prefix_v, [cand])
                    lo = jnp.where(pm <= p, cand, lo)
                pt = plsc.load_gather(prefix_v, [lo])
                sidx_v[i, pl.ds(h * _L, _L)] = (p - pt) * T + lo
            return carry

        lax.fori_loop(0, nchunks, gen_s, 0)

        def body(i, carry):
            for b in range(_NB):
                c = (b + 1) % _NB

                @pl.when(i % _NB == b)
                def _(b=b, c=c):
                    load(i, b, b).wait()
                    scat(i, b, b).start()

                    @pl.when(i + 1 < nchunks)
                    def _(b=b, c=c):
                        @pl.when(i >= _NB - 1)
                        def _(c=c):
                            scat(i - (_NB - 1), c, c).wait()

                            load(i + 1, c, c).start()

            return carry

        lax.fori_loop(0, nchunks, body, 0)

        for j in range(nchunks - _NB, nchunks):
            scat(j, j % _NB, j % _NB).wait()

        def zdrain(j, carry):
            pltpu.make_async_copy(zero_v, out_hbm.at[zidx_v.at[j]],
                                  zsem).wait()
            return carry

        lax.fori_loop(0, pchunks, zdrain, 0)

    return k


def kernel(x, lengths):
    N, D = x.shape
    B = lengths.shape[0]
    T = T_OUT
    P = B * T - N

    info = plsc.get_sparse_core_info()
    NC, NS = info.num_cores, info.num_subcores
    NW = NC * NS

    t = jnp.arange(T, dtype=jnp.int32)
    lens32 = lengths.astype(jnp.int32)
    prefix = jnp.sum(jnp.minimum(lens32[None, :], t[:, None]),
                     axis=1).astype(jnp.int32)
    i32 = jnp.arange(32, dtype=jnp.int32)
    cumpad = jnp.where(
        i32 > B,
        jnp.iinfo(jnp.int32).max,
        jnp.sum(jnp.where(jnp.arange(B, dtype=jnp.int32)[None, :]
                          < i32[:, None],
                          T - lens32[None, :], 0), axis=1)).astype(jnp.int32)
    zsrc = jnp.zeros((_CZ, D), x.dtype)

    k = _build_sc_kernel(N, P, D, T, NW, NC)
    out = k(x, prefix, cumpad, lens32, zsrc)
    return out.reshape(B, T, D)
